# baseline (speedup 1.0000x reference)
import functools

import jax
import jax.numpy as jnp
from jax import lax
from jax.experimental import pallas as pl
from jax.experimental.pallas import tpu as pltpu
from jax.experimental.pallas import tpu_sc as plsc


def kernel(uEmbeds, iEmbeds, ancs, poss, negs):
    B = ancs.shape[0]
    U, D = uEmbeds.shape
    I = iEmbeds.shape[0]
    rpl = 128 // D
    u2 = uEmbeds.reshape(U // rpl, 128)
    i2 = iEmbeds.reshape(I // rpl, 128)

    info = plsc.get_sparse_core_info()
    nc, ns = info.num_cores, info.num_subcores
    nw = nc * ns
    b_per_w = B // nw
    C = 256
    n_chunks = b_per_w // C
    mesh = plsc.VectorSubcoreMesh(core_axis_name="c", subcore_axis_name="s")

    @functools.partial(
        pl.kernel,
        mesh=mesh,
        out_type=jax.ShapeDtypeStruct((B,), jnp.float32),
        compiler_params=pltpu.CompilerParams(
            needs_layout_passes=False, use_tc_tiling_on_sc=True),
        scratch_types=[
            pltpu.VMEM((b_per_w,), jnp.int32),
            pltpu.VMEM((b_per_w,), jnp.int32),
            pltpu.VMEM((b_per_w,), jnp.int32),
            pltpu.VMEM((C,), jnp.int32),
            pltpu.VMEM((C,), jnp.int32),
            pltpu.VMEM((C,), jnp.int32),
            pltpu.VMEM((C, 128), jnp.float32),
            pltpu.VMEM((C, 128), jnp.float32),
            pltpu.VMEM((C, 128), jnp.float32),
            pltpu.VMEM((b_per_w,), jnp.float32),
            pltpu.SemaphoreType.DMA,
            pltpu.SemaphoreType.DMA,
            pltpu.SemaphoreType.DMA,
        ],
    )
    def mf_scores(u_hbm, i_hbm, anc_hbm, pos_hbm, neg_hbm, out_hbm,
                  ids_a, ids_p, ids_n, lin_a, lin_p, lin_n,
                  la, lp, ln, out_v, sem_a, sem_p, sem_n):
        wid = lax.axis_index("s") * nc + lax.axis_index("c")
        base = wid * b_per_w
        pltpu.sync_copy(anc_hbm.at[pl.ds(base, b_per_w)], ids_a)
        pltpu.sync_copy(pos_hbm.at[pl.ds(base, b_per_w)], ids_p)
        pltpu.sync_copy(neg_hbm.at[pl.ds(base, b_per_w)], ids_n)

        lane = lax.iota(jnp.int32, 16)

        for k in range(n_chunks):
            def line_prep(j, carry):
                s = k * C + j * 16
                d = j * 16
                lin_a[pl.ds(d, 16)] = lax.shift_right_logical(
                    ids_a[pl.ds(s, 16)], 3)
                lin_p[pl.ds(d, 16)] = lax.shift_right_logical(
                    ids_p[pl.ds(s, 16)], 3)
                lin_n[pl.ds(d, 16)] = lax.shift_right_logical(
                    ids_n[pl.ds(s, 16)], 3)
                return carry

            lax.fori_loop(0, C // 16, line_prep, 0)

            ca_ = pltpu.async_copy(u_hbm.at[lin_a], la, sem_a)
            cp_ = pltpu.async_copy(i_hbm.at[lin_p], lp, sem_p)
            cn_ = pltpu.async_copy(i_hbm.at[lin_n], ln, sem_n)
            ca_.wait()
            cp_.wait()
            cn_.wait()

            def dot_body(j, carry):
                s = k * C + j * 16
                rows = j * 16 + lane
                ca = (ids_a[pl.ds(s, 16)] & 7) * D
                cp = (ids_p[pl.ds(s, 16)] & 7) * D
                cn = (ids_n[pl.ds(s, 16)] & 7) * D
                acc = jnp.zeros((16,), jnp.float32)
                for col in range(D):
                    a = plsc.load_gather(la, [rows, ca + col])
                    p = plsc.load_gather(lp, [rows, cp + col])
                    n = plsc.load_gather(ln, [rows, cn + col])
                    acc = acc + a * (p - n)
                out_v[pl.ds(s, 16)] = acc
                return carry

            lax.fori_loop(0, C // 16, dot_body, 0)

        pltpu.sync_copy(out_v, out_hbm.at[pl.ds(base, b_per_w)])

    return mf_scores(u2, i2, ancs, poss, negs)


# speedup vs baseline: 4.3288x; 4.3288x over previous
import functools

import jax
import jax.numpy as jnp
from jax import lax
from jax.experimental import pallas as pl
from jax.experimental.pallas import tpu as pltpu
from jax.experimental.pallas import tpu_sc as plsc


def kernel(uEmbeds, iEmbeds, ancs, poss, negs):
    B = ancs.shape[0]
    D = uEmbeds.shape[1]
    uT = uEmbeds.T
    iT = iEmbeds.T

    info = plsc.get_sparse_core_info()
    nc, ns = info.num_cores, info.num_subcores
    nw = nc * ns
    b_per_w = B // nw
    mesh = plsc.VectorSubcoreMesh(core_axis_name="c", subcore_axis_name="s")

    @functools.partial(
        pl.kernel,
        mesh=mesh,
        out_type=jax.ShapeDtypeStruct((B,), jnp.float32),
        compiler_params=pltpu.CompilerParams(
            needs_layout_passes=False, use_tc_tiling_on_sc=True),
        scratch_types=[
            pltpu.VMEM((b_per_w,), jnp.int32),
            pltpu.VMEM((b_per_w,), jnp.int32),
            pltpu.VMEM((b_per_w,), jnp.int32),
            pltpu.VMEM((16 * D, 128), jnp.float32),
            pltpu.VMEM((16 * D, 128), jnp.float32),
            pltpu.VMEM((16 * D, 128), jnp.float32),
            pltpu.VMEM((b_per_w,), jnp.float32),
            pltpu.SemaphoreType.DMA,
            pltpu.SemaphoreType.DMA,
            pltpu.SemaphoreType.DMA,
        ],
    )
    def mf_scores(u_hbm, i_hbm, anc_hbm, pos_hbm, neg_hbm, out_hbm,
                  ids_a, ids_p, ids_n, la, lp, ln, out_v,
                  sem_a, sem_p, sem_n):
        wid = lax.axis_index("s") * nc + lax.axis_index("c")
        base = wid * b_per_w
        pltpu.sync_copy(anc_hbm.at[pl.ds(base, b_per_w)], ids_a)
        pltpu.sync_copy(pos_hbm.at[pl.ds(base, b_per_w)], ids_p)
        pltpu.sync_copy(neg_hbm.at[pl.ds(base, b_per_w)], ids_n)

        lane = lax.iota(jnp.int32, 16)

        def chunk_body(j, carry):
            s = j * 16
            va = ids_a[pl.ds(s, 16)]
            vp = ids_p[pl.ds(s, 16)]
            vn = ids_n[pl.ds(s, 16)]
            copies = []
            for k in range(16):
                ba = pl.multiple_of((va[k] >> 7) * 128, 128)
                bp = pl.multiple_of((vp[k] >> 7) * 128, 128)
                bn = pl.multiple_of((vn[k] >> 7) * 128, 128)
                copies.append(pltpu.async_copy(
                    u_hbm.at[:, pl.ds(ba, 128)],
                    la.at[pl.ds(k * D, D), :], sem_a))
                copies.append(pltpu.async_copy(
                    i_hbm.at[:, pl.ds(bp, 128)],
                    lp.at[pl.ds(k * D, D), :], sem_p))
                copies.append(pltpu.async_copy(
                    i_hbm.at[:, pl.ds(bn, 128)],
                    ln.at[pl.ds(k * D, D), :], sem_n))
            for c in copies:
                c.wait()
            pa = va & 127
            pp = vp & 127
            pn = vn & 127
            acc = jnp.zeros((16,), jnp.float32)
            for l in range(D):
                rows = lane * D + l
                a = plsc.load_gather(la, [rows, pa])
                p = plsc.load_gather(lp, [rows, pp])
                n = plsc.load_gather(ln, [rows, pn])
                acc = acc + a * (p - n)
            out_v[pl.ds(s, 16)] = acc
            return carry

        lax.fori_loop(0, b_per_w // 16, chunk_body, 0)
        pltpu.sync_copy(out_v, out_hbm.at[pl.ds(base, b_per_w)])

    return mf_scores(uT, iT, ancs, poss, negs)


# speedup vs baseline: 4.8751x; 1.1262x over previous
import functools

import jax
import jax.numpy as jnp
from jax import lax
from jax.experimental import pallas as pl
from jax.experimental.pallas import tpu as pltpu
from jax.experimental.pallas import tpu_sc as plsc


def kernel(uEmbeds, iEmbeds, ancs, poss, negs):
    B = ancs.shape[0]
    D = uEmbeds.shape[1]
    uT = uEmbeds.T
    iT = iEmbeds.T

    CH = 8
    NBUF = 2

    info = plsc.get_sparse_core_info()
    nc, ns = info.num_cores, info.num_subcores
    nw = nc * ns
    b_per_w = B // nw
    n_chunks = b_per_w // CH
    mesh = plsc.VectorSubcoreMesh(core_axis_name="c", subcore_axis_name="s")

    @functools.partial(
        pl.kernel,
        mesh=mesh,
        out_type=jax.ShapeDtypeStruct((B,), jnp.float32),
        compiler_params=pltpu.CompilerParams(
            needs_layout_passes=False, use_tc_tiling_on_sc=True),
        scratch_types=[
            pltpu.VMEM((b_per_w + 16,), jnp.int32),
            pltpu.VMEM((b_per_w + 16,), jnp.int32),
            pltpu.VMEM((b_per_w + 16,), jnp.int32),
            pltpu.VMEM((NBUF, CH * D, 128), jnp.float32),
            pltpu.VMEM((NBUF, CH * D, 128), jnp.float32),
            pltpu.VMEM((NBUF, CH * D, 128), jnp.float32),
            pltpu.VMEM((b_per_w,), jnp.float32),
            pltpu.SemaphoreType.DMA,
            pltpu.SemaphoreType.DMA,
            pltpu.SemaphoreType.DMA,
        ],
    )
    def mf_scores(u_hbm, i_hbm, anc_hbm, pos_hbm, neg_hbm, out_hbm,
                  ids_a, ids_p, ids_n, la, lp, ln, out_v,
                  sem_a, sem_p, sem_n):
        wid = lax.axis_index("s") * nc + lax.axis_index("c")
        base = wid * b_per_w
        pltpu.sync_copy(anc_hbm.at[pl.ds(base, b_per_w)], ids_a.at[pl.ds(0, b_per_w)])
        pltpu.sync_copy(pos_hbm.at[pl.ds(base, b_per_w)], ids_p.at[pl.ds(0, b_per_w)])
        pltpu.sync_copy(neg_hbm.at[pl.ds(base, b_per_w)], ids_n.at[pl.ds(0, b_per_w)])
        zeros16 = jnp.zeros((16,), jnp.int32)
        ids_a[pl.ds(b_per_w, 16)] = zeros16
        ids_p[pl.ds(b_per_w, 16)] = zeros16
        ids_n[pl.ds(b_per_w, 16)] = zeros16

        lane = lax.iota(jnp.int32, 16)
        low = lane < CH

        def fire(c, q):
            s = c * CH
            va = ids_a[pl.ds(s, 16)]
            vp = ids_p[pl.ds(s, 16)]
            vn = ids_n[pl.ds(s, 16)]
            for k in range(CH):
                ba = pl.multiple_of((va[k] >> 7) * 128, 128)
                bp = pl.multiple_of((vp[k] >> 7) * 128, 128)
                bn = pl.multiple_of((vn[k] >> 7) * 128, 128)
                pltpu.async_copy(
                    u_hbm.at[:, pl.ds(ba, 128)],
                    la.at[q, pl.ds(k * D, D), :], sem_a)
                pltpu.async_copy(
                    i_hbm.at[:, pl.ds(bp, 128)],
                    lp.at[q, pl.ds(k * D, D), :], sem_p)
                pltpu.async_copy(
                    i_hbm.at[:, pl.ds(bn, 128)],
                    ln.at[q, pl.ds(k * D, D), :], sem_n)

        def drain(q):
            for k in range(CH):
                pltpu.make_async_copy(
                    u_hbm.at[:, pl.ds(0, 128)],
                    la.at[q, pl.ds(k * D, D), :], sem_a).wait()
                pltpu.make_async_copy(
                    i_hbm.at[:, pl.ds(0, 128)],
                    lp.at[q, pl.ds(k * D, D), :], sem_p).wait()
                pltpu.make_async_copy(
                    i_hbm.at[:, pl.ds(0, 128)],
                    ln.at[q, pl.ds(k * D, D), :], sem_n).wait()

        def extract(c, q):
            s = c * CH
            va = ids_a[pl.ds(s, 16)]
            vp = ids_p[pl.ds(s, 16)]
            vn = ids_n[pl.ds(s, 16)]
            pa = va & 127
            pp = vp & 127
            pn = vn & 127
            slot = jnp.where(low, lane, 0) * D
            acc = jnp.zeros((16,), jnp.float32)
            for l in range(D):
                rows = slot + l
                a = plsc.load_gather(la.at[q], [rows, pa])
                p = plsc.load_gather(lp.at[q], [rows, pp])
                n = plsc.load_gather(ln.at[q], [rows, pn])
                acc = acc + a * (p - n)
            plsc.store_scatter(out_v, [s + lane], acc, mask=low)

        fire(0, 0)

        def pair_body(j2, carry):
            e = j2 * 2
            fire(e + 1, 1)
            drain(0)
            extract(e, 0)

            @pl.when(j2 < n_chunks // 2 - 1)
            def _():
                fire(e + 2, 0)

            drain(1)
            extract(e + 1, 1)
            return carry

        lax.fori_loop(0, n_chunks // 2, pair_body, 0)
        pltpu.sync_copy(out_v, out_hbm.at[pl.ds(base, b_per_w)])

    return mf_scores(uT, iT, ancs, poss, negs)


# speedup vs baseline: 4.8819x; 1.0014x over previous
import functools

import jax
import jax.numpy as jnp
from jax import lax
from jax.experimental import pallas as pl
from jax.experimental.pallas import tpu as pltpu
from jax.experimental.pallas import tpu_sc as plsc


def kernel(uEmbeds, iEmbeds, ancs, poss, negs):
    B = ancs.shape[0]
    D = uEmbeds.shape[1]
    uT = uEmbeds.T
    iT = iEmbeds.T

    CH = 8
    NBUF = 2

    info = plsc.get_sparse_core_info()
    nc, ns = info.num_cores, info.num_subcores
    nw = nc * ns
    b_per_w = B // nw
    n_chunks = b_per_w // CH
    mesh = plsc.VectorSubcoreMesh(core_axis_name="c", subcore_axis_name="s")

    @functools.partial(
        pl.kernel,
        mesh=mesh,
        out_type=jax.ShapeDtypeStruct((B,), jnp.float32),
        compiler_params=pltpu.CompilerParams(
            needs_layout_passes=False, use_tc_tiling_on_sc=True),
        scratch_types=[
            pltpu.VMEM((b_per_w + 16,), jnp.int32),
            pltpu.VMEM((b_per_w + 16,), jnp.int32),
            pltpu.VMEM((b_per_w + 16,), jnp.int32),
            pltpu.VMEM((NBUF, CH * D, 128), jnp.float32),
            pltpu.VMEM((NBUF, CH * D, 128), jnp.float32),
            pltpu.VMEM((NBUF, CH * D, 128), jnp.float32),
            pltpu.VMEM((b_per_w,), jnp.float32),
            pltpu.SemaphoreType.DMA,
            pltpu.SemaphoreType.DMA,
            pltpu.SemaphoreType.DMA,
            pltpu.SemaphoreType.DMA,
            pltpu.SemaphoreType.DMA,
            pltpu.SemaphoreType.DMA,
        ],
    )
    def mf_scores(u_hbm, i_hbm, anc_hbm, pos_hbm, neg_hbm, out_hbm,
                  ids_a, ids_p, ids_n, la, lp, ln, out_v,
                  sem_a0, sem_p0, sem_n0, sem_a1, sem_p1, sem_n1):
        sems = ((sem_a0, sem_p0, sem_n0), (sem_a1, sem_p1, sem_n1))
        wid = lax.axis_index("s") * nc + lax.axis_index("c")
        base = wid * b_per_w
        pltpu.sync_copy(anc_hbm.at[pl.ds(base, b_per_w)], ids_a.at[pl.ds(0, b_per_w)])
        pltpu.sync_copy(pos_hbm.at[pl.ds(base, b_per_w)], ids_p.at[pl.ds(0, b_per_w)])
        pltpu.sync_copy(neg_hbm.at[pl.ds(base, b_per_w)], ids_n.at[pl.ds(0, b_per_w)])
        zeros16 = jnp.zeros((16,), jnp.int32)
        ids_a[pl.ds(b_per_w, 16)] = zeros16
        ids_p[pl.ds(b_per_w, 16)] = zeros16
        ids_n[pl.ds(b_per_w, 16)] = zeros16

        lane = lax.iota(jnp.int32, 16)
        low = lane < CH

        def fire(c, q):
            s = c * CH
            va = ids_a[pl.ds(s, 16)]
            vp = ids_p[pl.ds(s, 16)]
            vn = ids_n[pl.ds(s, 16)]
            sa, sp, sn = sems[q]
            for k in range(CH):
                ba = pl.multiple_of((va[k] >> 7) * 128, 128)
                bp = pl.multiple_of((vp[k] >> 7) * 128, 128)
                bn = pl.multiple_of((vn[k] >> 7) * 128, 128)
                pltpu.async_copy(
                    u_hbm.at[:, pl.ds(ba, 128)],
                    la.at[q, pl.ds(k * D, D), :], sa)
                pltpu.async_copy(
                    i_hbm.at[:, pl.ds(bp, 128)],
                    lp.at[q, pl.ds(k * D, D), :], sp)
                pltpu.async_copy(
                    i_hbm.at[:, pl.ds(bn, 128)],
                    ln.at[q, pl.ds(k * D, D), :], sn)

        def drain(q):
            sa, sp, sn = sems[q]
            for k in range(CH):
                pltpu.make_async_copy(
                    u_hbm.at[:, pl.ds(0, 128)],
                    la.at[q, pl.ds(k * D, D), :], sa).wait()
                pltpu.make_async_copy(
                    i_hbm.at[:, pl.ds(0, 128)],
                    lp.at[q, pl.ds(k * D, D), :], sp).wait()
                pltpu.make_async_copy(
                    i_hbm.at[:, pl.ds(0, 128)],
                    ln.at[q, pl.ds(k * D, D), :], sn).wait()

        def extract(c, q):
            s = c * CH
            va = ids_a[pl.ds(s, 16)]
            vp = ids_p[pl.ds(s, 16)]
            vn = ids_n[pl.ds(s, 16)]
            pa = va & 127
            pp = vp & 127
            pn = vn & 127
            slot = jnp.where(low, lane, 0) * D
            acc = jnp.zeros((16,), jnp.float32)
            for l in range(D):
                rows = slot + l
                a = plsc.load_gather(la.at[q], [rows, pa])
                p = plsc.load_gather(lp.at[q], [rows, pp])
                n = plsc.load_gather(ln.at[q], [rows, pn])
                acc = acc + a * (p - n)
            plsc.store_scatter(out_v, [s + lane], acc, mask=low)

        fire(0, 0)

        def pair_body(j2, carry):
            e = j2 * 2
            fire(e + 1, 1)
            drain(0)
            extract(e, 0)

            @pl.when(j2 < n_chunks // 2 - 1)
            def _():
                fire(e + 2, 0)

            drain(1)
            extract(e + 1, 1)
            return carry

        lax.fori_loop(0, n_chunks // 2, pair_body, 0)
        pltpu.sync_copy(out_v, out_hbm.at[pl.ds(base, b_per_w)])

    return mf_scores(uT, iT, ancs, poss, negs)
